# speedup vs baseline: 3.3366x; 3.3366x over previous
import functools

import jax
import jax.numpy as jnp
from jax import lax
from jax.experimental import pallas as pl
from jax.experimental.pallas import tpu as pltpu
from jax.experimental.pallas import tpu_sc as plsc

_NC = 2
_NS = 16
_NW = _NC * _NS
_CHUNK = 128


@functools.lru_cache(maxsize=None)
def _build(n_chunks: int, d: int):
    b_total = _NW * n_chunks * _CHUNK
    mesh = plsc.VectorSubcoreMesh(
        core_axis_name="c", subcore_axis_name="s",
        num_cores=_NC, num_subcores=_NS,
    )

    @functools.partial(
        pl.kernel,
        out_type=jax.ShapeDtypeStruct((b_total, d), jnp.float32),
        mesh=mesh,
        scratch_types=[
            pltpu.VMEM((n_chunks, _CHUNK), jnp.int32),
            pltpu.VMEM((2, _CHUNK, d), jnp.float32),
            pltpu.SemaphoreType.DMA,
            pltpu.SemaphoreType.DMA,
        ],
    )
    def embed(table_hbm, idx_hbm, out_hbm, idx_v, rows_v, sem0, sem1):
        sems = (sem0, sem1)
        wid = lax.axis_index("s") * _NC + lax.axis_index("c")
        base = wid * (n_chunks * _CHUNK)
        pltpu.sync_copy(idx_hbm.at[wid], idx_v)
        pltpu.async_copy(table_hbm.at[idx_v.at[0]], rows_v.at[0], sem0)
        pltpu.async_copy(table_hbm.at[idx_v.at[1]], rows_v.at[1], sem1)

        @pl.loop(0, n_chunks, step=2)
        def _(j):
            for b in range(2):
                jj = j + b
                pltpu.make_async_copy(
                    table_hbm.at[idx_v.at[jj]], rows_v.at[b], sems[b]
                ).wait()
                pltpu.sync_copy(
                    rows_v.at[b], out_hbm.at[pl.ds(base + jj * _CHUNK, _CHUNK)]
                )
                nxt = jj + 2

                @pl.when(nxt < n_chunks)
                def _():
                    pltpu.async_copy(
                        table_hbm.at[idx_v.at[nxt]], rows_v.at[b], sems[b]
                    )

    return embed


def kernel(x, table):
    bt, h = x.shape
    v, d = table.shape
    b_total = bt * h
    n_chunks = b_total // (_NW * _CHUNK)
    idx = x.reshape(_NW, n_chunks, _CHUNK).astype(jnp.int32)
    out = _build(n_chunks, d)(table, idx)
    return out.reshape(bt, h, d)


# speedup vs baseline: 3.3451x; 1.0026x over previous
import functools

import jax
import jax.numpy as jnp
from jax import lax
from jax.experimental import pallas as pl
from jax.experimental.pallas import tpu as pltpu
from jax.experimental.pallas import tpu_sc as plsc

_NC = 2
_NS = 16
_NW = _NC * _NS
_CHUNK = 128
_NBUF = 4


@functools.lru_cache(maxsize=None)
def _build(n_chunks: int, d: int):
    assert n_chunks >= 6 and (n_chunks - 6) % _NBUF == 0
    b_total = _NW * n_chunks * _CHUNK
    mesh = plsc.VectorSubcoreMesh(
        core_axis_name="c", subcore_axis_name="s",
        num_cores=_NC, num_subcores=_NS,
    )

    @functools.partial(
        pl.kernel,
        out_type=jax.ShapeDtypeStruct((b_total, d), jnp.float32),
        mesh=mesh,
        scratch_types=[
            pltpu.VMEM((n_chunks, _CHUNK), jnp.int32),
            pltpu.VMEM((_NBUF, _CHUNK, d), jnp.float32),
            [pltpu.SemaphoreType.DMA] * _NBUF,
            [pltpu.SemaphoreType.DMA] * _NBUF,
        ],
    )
    def embed(table_hbm, idx_hbm, out_hbm, idx_v, rows_v, gsems, wsems):
        wid = lax.axis_index("s") * _NC + lax.axis_index("c")
        base = wid * (n_chunks * _CHUNK)
        pltpu.sync_copy(idx_hbm.at[wid], idx_v)

        def issue_g(jj, b):
            pltpu.async_copy(table_hbm.at[idx_v.at[jj]], rows_v.at[b], gsems[b])

        def wait_g(b):
            pltpu.make_async_copy(
                table_hbm.at[idx_v.at[0]], rows_v.at[b], gsems[b]
            ).wait()

        def issue_w(jj, b):
            pltpu.async_copy(
                rows_v.at[b], out_hbm.at[pl.ds(base + jj * _CHUNK, _CHUNK)],
                wsems[b],
            )

        def wait_w(b):
            pltpu.make_async_copy(
                rows_v.at[b], out_hbm.at[pl.ds(base, _CHUNK)], wsems[b]
            ).wait()

        for c in range(_NBUF):
            issue_g(c, c)
        for c in range(2):
            wait_g(c)
            issue_w(c, c)

        @pl.loop(2, n_chunks - 4, step=_NBUF)
        def _(j):
            for b in range(_NBUF):
                jj = j + b
                b_refill = b
                b_cur = (b + 2) % _NBUF
                wait_w(b_refill)
                issue_g(jj + 2, b_refill)
                wait_g(b_cur)
                issue_w(jj, b_cur)

        for jj in range(n_chunks - 4, n_chunks):
            wait_w((jj - 2) % _NBUF)
            if jj + 2 < n_chunks:
                issue_g(jj + 2, (jj - 2) % _NBUF)
            wait_g(jj % _NBUF)
            issue_w(jj, jj % _NBUF)
        wait_w((n_chunks - 2) % _NBUF)
        wait_w((n_chunks - 1) % _NBUF)

    return embed


def kernel(x, table):
    bt, h = x.shape
    v, d = table.shape
    b_total = bt * h
    n_chunks = b_total // (_NW * _CHUNK)
    idx = x.reshape(_NW, n_chunks, _CHUNK).astype(jnp.int32)
    out = _build(n_chunks, d)(table, idx)
    return out.reshape(bt, h, d)


# speedup vs baseline: 5.7642x; 1.7232x over previous
import functools

import jax
import jax.numpy as jnp
from jax import lax
from jax.experimental import pallas as pl
from jax.experimental.pallas import tpu as pltpu
from jax.experimental.pallas import tpu_sc as plsc

_NC = 2
_NS = 16
_NW = _NC * _NS
_NBUF = 4
_HPAD = 56
_PAIR = 2


@functools.lru_cache(maxsize=None)
def _build(bt: int, h: int, d: int):
    b_per_w = bt // _NW
    steps = b_per_w // _PAIR
    chunk = _PAIR * _HPAD
    assert steps >= 6 and steps % 4 == 0
    mesh = plsc.VectorSubcoreMesh(
        core_axis_name="c", subcore_axis_name="s",
        num_cores=_NC, num_subcores=_NS,
    )

    @functools.partial(
        pl.kernel,
        out_type=jax.ShapeDtypeStruct((bt, h, d), jnp.float32),
        mesh=mesh,
        compiler_params=pltpu.CompilerParams(use_tc_tiling_on_sc=True),
        scratch_types=[
            pltpu.VMEM((b_per_w * _HPAD,), jnp.int32),
            pltpu.VMEM((_NBUF, chunk, d), jnp.float32),
            [pltpu.SemaphoreType.DMA] * _NBUF,
            [pltpu.SemaphoreType.DMA] * _NBUF,
        ],
    )
    def embed(table_hbm, idx_hbm, out_hbm, idx_v, rows_v, gsems, wsems):
        wid = lax.axis_index("s") * _NC + lax.axis_index("c")
        base = wid * b_per_w
        pltpu.sync_copy(
            idx_hbm.at[pl.ds(wid * (b_per_w * _HPAD), b_per_w * _HPAD)], idx_v
        )

        def issue_g(p, b):
            pltpu.async_copy(
                table_hbm.at[idx_v.at[pl.ds(p * chunk, chunk)]],
                rows_v.at[b], gsems[b],
            )

        def wait_g(b):
            pltpu.make_async_copy(
                table_hbm.at[idx_v.at[pl.ds(0, chunk)]], rows_v.at[b], gsems[b]
            ).wait()

        def issue_w(p, b):
            b0 = base + p * _PAIR
            pltpu.async_copy(
                rows_v.at[b].at[pl.ds(0, h)], out_hbm.at[b0], wsems[b]
            )
            pltpu.async_copy(
                rows_v.at[b].at[pl.ds(_HPAD, h)], out_hbm.at[b0 + 1], wsems[b]
            )

        def wait_w(b):
            for _ in range(_PAIR):
                pltpu.make_async_copy(
                    rows_v.at[b].at[pl.ds(0, h)], out_hbm.at[0], wsems[b]
                ).wait()

        for c in range(_NBUF):
            issue_g(c, c)
        for c in range(2):
            wait_g(c)
            issue_w(c, c)

        @pl.loop(2, steps - 2, step=_NBUF)
        def _(j):
            for b in range(_NBUF):
                p = j + b
                b_refill = b
                b_cur = (b + 2) % _NBUF
                wait_w(b_refill)
                issue_g(p + 2, b_refill)
                wait_g(b_cur)
                issue_w(p, b_cur)

        for p in range(steps - 2, steps):
            wait_w((p - 2) % _NBUF)
            wait_g(p % _NBUF)
            issue_w(p, p % _NBUF)
        wait_w((steps - 2) % _NBUF)
        wait_w((steps - 1) % _NBUF)

    return embed


def kernel(x, table):
    bt, h = x.shape
    v, d = table.shape
    xp = jnp.concatenate([x, x[:, h - (_HPAD - h):]], axis=1)
    idx = xp.reshape(-1).astype(jnp.int32)
    return _build(bt, h, d)(table, idx)


# speedup vs baseline: 10.4321x; 1.8098x over previous
import functools

import jax
import jax.numpy as jnp
from jax import lax
from jax.experimental import pallas as pl
from jax.experimental.pallas import tpu as pltpu
from jax.experimental.pallas import tpu_sc as plsc

_NC = 2
_NS = 16
_NW = _NC * _NS
_CHUNK = 128
_NBUF = 4


@functools.lru_cache(maxsize=None)
def _build(n_chunks: int, d: int):
    assert n_chunks >= 6 and (n_chunks - 6) % _NBUF == 0
    b_total = _NW * n_chunks * _CHUNK
    rows_per_w = n_chunks * _CHUNK
    mesh = plsc.VectorSubcoreMesh(
        core_axis_name="c", subcore_axis_name="s",
        num_cores=_NC, num_subcores=_NS,
    )

    @functools.partial(
        pl.kernel,
        out_type=jax.ShapeDtypeStruct((b_total, d), jnp.float32),
        mesh=mesh,
        compiler_params=pltpu.CompilerParams(use_tc_tiling_on_sc=True),
        scratch_types=[
            pltpu.VMEM((rows_per_w,), jnp.int32),
            pltpu.VMEM((_NBUF, _CHUNK, d), jnp.float32),
            [pltpu.SemaphoreType.DMA] * _NBUF,
            [pltpu.SemaphoreType.DMA] * _NBUF,
        ],
    )
    def embed(table_hbm, idx_hbm, out_hbm, idx_v, rows_v, gsems, wsems):
        wid = lax.axis_index("s") * _NC + lax.axis_index("c")
        base = wid * rows_per_w
        pltpu.sync_copy(idx_hbm.at[pl.ds(base, rows_per_w)], idx_v)

        def issue_g(jj, b):
            pltpu.async_copy(
                table_hbm.at[idx_v.at[pl.ds(jj * _CHUNK, _CHUNK)]],
                rows_v.at[b], gsems[b],
            )

        def wait_g(b):
            pltpu.make_async_copy(
                table_hbm.at[idx_v.at[pl.ds(0, _CHUNK)]], rows_v.at[b],
                gsems[b],
            ).wait()

        def issue_w(jj, b):
            pltpu.async_copy(
                rows_v.at[b], out_hbm.at[pl.ds(base + jj * _CHUNK, _CHUNK)],
                wsems[b],
            )

        def wait_w(b):
            pltpu.make_async_copy(
                rows_v.at[b], out_hbm.at[pl.ds(base, _CHUNK)], wsems[b]
            ).wait()

        for c in range(_NBUF):
            issue_g(c, c)
        for c in range(2):
            wait_g(c)
            issue_w(c, c)

        @pl.loop(2, n_chunks - 4, step=_NBUF)
        def _(j):
            for b in range(_NBUF):
                jj = j + b
                b_refill = b
                b_cur = (b + 2) % _NBUF
                wait_w(b_refill)
                issue_g(jj + 2, b_refill)
                wait_g(b_cur)
                issue_w(jj, b_cur)

        for jj in range(n_chunks - 4, n_chunks):
            wait_w((jj - 2) % _NBUF)
            if jj + 2 < n_chunks:
                issue_g(jj + 2, (jj - 2) % _NBUF)
            wait_g(jj % _NBUF)
            issue_w(jj, jj % _NBUF)
        wait_w((n_chunks - 2) % _NBUF)
        wait_w((n_chunks - 1) % _NBUF)

    return embed


def kernel(x, table):
    bt, h = x.shape
    v, d = table.shape
    b_total = bt * h
    n_chunks = b_total // (_NW * _CHUNK)
    idx = x.T.reshape(-1).astype(jnp.int32)
    out = _build(n_chunks, d)(table, idx)
    return out.reshape(h, bt, d).transpose(1, 0, 2)


# speedup vs baseline: 10.5662x; 1.0128x over previous
import functools

import jax
import jax.numpy as jnp
from jax import lax
from jax.experimental import pallas as pl
from jax.experimental.pallas import tpu as pltpu
from jax.experimental.pallas import tpu_sc as plsc

_NC = 2
_NS = 16
_NW = _NC * _NS
_CHUNK = 128
_NBUF = 6
_PF = 4


@functools.lru_cache(maxsize=None)
def _build(n_chunks: int, d: int):
    assert n_chunks >= 8 and (n_chunks - 8) % _NBUF == 0
    b_total = _NW * n_chunks * _CHUNK
    rows_per_w = n_chunks * _CHUNK
    mesh = plsc.VectorSubcoreMesh(
        core_axis_name="c", subcore_axis_name="s",
        num_cores=_NC, num_subcores=_NS,
    )

    @functools.partial(
        pl.kernel,
        out_type=jax.ShapeDtypeStruct((b_total, d), jnp.float32),
        mesh=mesh,
        compiler_params=pltpu.CompilerParams(use_tc_tiling_on_sc=True),
        scratch_types=[
            pltpu.VMEM((rows_per_w,), jnp.int32),
            pltpu.VMEM((_NBUF, _CHUNK, d), jnp.float32),
            [pltpu.SemaphoreType.DMA] * _NBUF,
            [pltpu.SemaphoreType.DMA] * _NBUF,
        ],
    )
    def embed(table_hbm, idx_hbm, out_hbm, idx_v, rows_v, gsems, wsems):
        wid = lax.axis_index("s") * _NC + lax.axis_index("c")
        base = wid * rows_per_w
        pltpu.sync_copy(idx_hbm.at[pl.ds(base, rows_per_w)], idx_v)

        def issue_g(jj, b):
            pltpu.async_copy(
                table_hbm.at[idx_v.at[pl.ds(jj * _CHUNK, _CHUNK)]],
                rows_v.at[b], gsems[b],
            )

        def wait_g(b):
            pltpu.make_async_copy(
                table_hbm.at[idx_v.at[pl.ds(0, _CHUNK)]], rows_v.at[b],
                gsems[b],
            ).wait()

        def issue_w(jj, b):
            pltpu.async_copy(
                rows_v.at[b], out_hbm.at[pl.ds(base + jj * _CHUNK, _CHUNK)],
                wsems[b],
            )

        def wait_w(b):
            pltpu.make_async_copy(
                rows_v.at[b], out_hbm.at[pl.ds(base, _CHUNK)], wsems[b]
            ).wait()

        for c in range(_PF):
            issue_g(c, c)
        for c in range(2):
            issue_g(c + _PF, (c + _PF) % _NBUF)
            wait_g(c)
            issue_w(c, c)

        @pl.loop(2, n_chunks - _NBUF, step=_NBUF)
        def _(j):
            for b in range(_NBUF):
                jj = j + b
                b_refill = (b + 2 + _PF) % _NBUF
                b_cur = (b + 2) % _NBUF
                wait_w(b_refill)
                issue_g(jj + _PF, b_refill)
                wait_g(b_cur)
                issue_w(jj, b_cur)

        for jj in range(n_chunks - _NBUF, n_chunks):
            wait_w((jj + _PF) % _NBUF)
            if jj + _PF < n_chunks:
                issue_g(jj + _PF, (jj + _PF) % _NBUF)
            wait_g(jj % _NBUF)
            issue_w(jj, jj % _NBUF)
        for jj in range(n_chunks - 2, n_chunks):
            wait_w(jj % _NBUF)

    return embed


def kernel(x, table):
    bt, h = x.shape
    v, d = table.shape
    b_total = bt * h
    n_chunks = b_total // (_NW * _CHUNK)
    idx = x.T.reshape(-1).astype(jnp.int32)
    out = _build(n_chunks, d)(table, idx)
    return out.reshape(h, bt, d).transpose(1, 0, 2)
